# baseline (speedup 1.0000x reference)
import functools

import jax
import jax.numpy as jnp
from jax import lax
from jax.experimental import pallas as pl
from jax.experimental.pallas import tpu as pltpu
from jax.experimental.pallas import tpu_sc as plsc

NUM_CORES = 2
NUM_SUBCORES = 16
NW = NUM_CORES * NUM_SUBCORES

CHUNK = 800
NBUF = 2


def _emb_kernel(n_chunks, d, idx_hbm, table_hbm, out_hbm,
                idx_v, rows_v, sems_g, sems_w):
    wid = lax.axis_index("s") * NUM_CORES + lax.axis_index("c")
    row_base = wid * (n_chunks * CHUNK)

    pltpu.sync_copy(idx_hbm.at[wid], idx_v)

    def gather_desc(i, s):
        return pltpu.make_async_copy(table_hbm.at[idx_v.at[i]], rows_v[s],
                                     sems_g[s])

    def write_desc(i, s):
        return pltpu.make_async_copy(rows_v[s],
                                     out_hbm.at[pl.ds(row_base + i * CHUNK,
                                                      CHUNK)],
                                     sems_w[s])

    for s in range(NBUF):
        gather_desc(s, s).start()

    def body(j, _):
        for s in range(NBUF):
            i = j * NBUF + s
            gather_desc(i - NBUF, s).wait()
            write_desc(i - NBUF, s).start()
            write_desc(i - NBUF, s).wait()
            gather_desc(i, s).start()
        return _

    lax.fori_loop(1, n_chunks // NBUF, body, None)

    for s in range(NBUF):
        i = n_chunks - NBUF + s
        gather_desc(i, s).wait()
        write_desc(i, s).start()
    for s in range(NBUF):
        i = n_chunks - NBUF + s
        write_desc(i, s).wait()


def kernel(token_ids, weight):
    batch, seq = token_ids.shape
    n, d = weight.shape
    b = batch * seq
    assert b % (NW * CHUNK) == 0
    n_chunks = b // (NW * CHUNK)
    assert n_chunks % NBUF == 0

    flat_ids = token_ids.reshape(NW, n_chunks, CHUNK).astype(jnp.int32)

    mesh = plsc.VectorSubcoreMesh(
        core_axis_name="c", subcore_axis_name="s",
        num_cores=NUM_CORES, num_subcores=NUM_SUBCORES)

    run = pl.kernel(
        functools.partial(_emb_kernel, n_chunks, d),
        out_type=jax.ShapeDtypeStruct((b, d), jnp.float32),
        mesh=mesh,
        scratch_types=[
            pltpu.VMEM((n_chunks, CHUNK), jnp.int32),
            [pltpu.VMEM((CHUNK, d), jnp.float32) for _ in range(NBUF)],
            [pltpu.SemaphoreType.DMA for _ in range(NBUF)],
            [pltpu.SemaphoreType.DMA for _ in range(NBUF)],
        ],
        compiler_params=pltpu.CompilerParams(use_tc_tiling_on_sc=False),
    )
    return run(flat_ids, weight).reshape(batch, seq, d)


# speedup vs baseline: 1.0006x; 1.0006x over previous
import functools

import jax
import jax.numpy as jnp
from jax import lax
from jax.experimental import pallas as pl
from jax.experimental.pallas import tpu as pltpu
from jax.experimental.pallas import tpu_sc as plsc

NUM_CORES = 2
NUM_SUBCORES = 16
NW = NUM_CORES * NUM_SUBCORES

CHUNK = 800
NBUF = 2


def _emb_kernel(n_chunks, d, idx_hbm, table_hbm, out_hbm,
                idx_v, rows_v, sems_g, sems_w):
    wid = lax.axis_index("s") * NUM_CORES + lax.axis_index("c")
    row_base = wid * (n_chunks * CHUNK)

    pltpu.sync_copy(idx_hbm.at[wid], idx_v)

    def gather_desc(i, s):
        return pltpu.make_async_copy(table_hbm.at[idx_v.at[i]], rows_v[s],
                                     sems_g[s])

    def write_desc(i, s):
        return pltpu.make_async_copy(rows_v[s],
                                     out_hbm.at[pl.ds(row_base + i * CHUNK,
                                                      CHUNK)],
                                     sems_w[s])

    for s in range(NBUF):
        gather_desc(s, s).start()

    def body(j, _):
        for s in range(NBUF):
            i = j * NBUF + s
            gather_desc(i - NBUF, s).wait()
            write_desc(i - NBUF, s).start()
            write_desc(i - NBUF, s).wait()
            gather_desc(i, s).start()
        return _

    lax.fori_loop(1, n_chunks // NBUF, body, None)

    for s in range(NBUF):
        i = n_chunks - NBUF + s
        gather_desc(i, s).wait()
        write_desc(i, s).start()
    for s in range(NBUF):
        i = n_chunks - NBUF + s
        write_desc(i, s).wait()


def kernel(token_ids, weight):
    batch, seq = token_ids.shape
    n, d = weight.shape
    b = batch * seq
    assert b % (NW * CHUNK) == 0
    n_chunks = b // (NW * CHUNK)
    assert n_chunks % NBUF == 0

    flat_ids = token_ids.reshape(NW, n_chunks, CHUNK).astype(jnp.int32)

    w2d = lax.optimization_barrier(weight.reshape(-1)).reshape(n, d)

    mesh = plsc.VectorSubcoreMesh(
        core_axis_name="c", subcore_axis_name="s",
        num_cores=NUM_CORES, num_subcores=NUM_SUBCORES)

    run = pl.kernel(
        functools.partial(_emb_kernel, n_chunks, d),
        out_type=jax.ShapeDtypeStruct((b, d), jnp.float32),
        mesh=mesh,
        scratch_types=[
            pltpu.VMEM((n_chunks, CHUNK), jnp.int32),
            [pltpu.VMEM((CHUNK, d), jnp.float32) for _ in range(NBUF)],
            [pltpu.SemaphoreType.DMA for _ in range(NBUF)],
            [pltpu.SemaphoreType.DMA for _ in range(NBUF)],
        ],
        compiler_params=pltpu.CompilerParams(use_tc_tiling_on_sc=False),
    )
    out = lax.optimization_barrier(run(flat_ids, w2d).reshape(-1))
    return out.reshape(batch, seq, d)


# speedup vs baseline: 1.2197x; 1.2190x over previous
import functools

import jax
import jax.numpy as jnp
from jax import lax
from jax.experimental import pallas as pl
from jax.experimental.pallas import tpu as pltpu
from jax.experimental.pallas import tpu_sc as plsc

NUM_CORES = 2
NUM_SUBCORES = 16
NW = NUM_CORES * NUM_SUBCORES

CHUNK = 400
NBUF = 2

PAD_D = 128


def _emb_kernel(n_chunks, idx_hbm, table_hbm, out_hbm,
                idx_v, rows_v, sems_g, sems_w):
    wid = lax.axis_index("s") * NUM_CORES + lax.axis_index("c")
    row_base = wid * (n_chunks * CHUNK)

    pltpu.sync_copy(idx_hbm.at[wid], idx_v)

    def gather_desc(i, s):
        return pltpu.make_async_copy(table_hbm.at[idx_v.at[i]], rows_v[s],
                                     sems_g[s])

    def write_desc(i, s):
        return pltpu.make_async_copy(rows_v[s],
                                     out_hbm.at[pl.ds(row_base + i * CHUNK,
                                                      CHUNK)],
                                     sems_w[s])

    for s in range(NBUF):
        gather_desc(s, s).start()

    def body(j, _):
        for s in range(NBUF):
            i = j * NBUF + s
            gather_desc(i - NBUF, s).wait()
            write_desc(i - NBUF, s).start()
            write_desc(i - NBUF, s).wait()
            gather_desc(i, s).start()
        return _

    lax.fori_loop(1, n_chunks // NBUF, body, None)

    for s in range(NBUF):
        i = n_chunks - NBUF + s
        gather_desc(i, s).wait()
        write_desc(i, s).start()
    for s in range(NBUF):
        i = n_chunks - NBUF + s
        write_desc(i, s).wait()


def kernel(token_ids, weight):
    batch, seq = token_ids.shape
    n, d = weight.shape
    b = batch * seq
    assert b % (NW * CHUNK) == 0
    n_chunks = b // (NW * CHUNK)
    assert n_chunks % NBUF == 0

    flat_ids = token_ids.reshape(NW, n_chunks, CHUNK).astype(jnp.int32)
    wpad = jnp.pad(weight, ((0, 0), (0, PAD_D - d)))

    mesh = plsc.VectorSubcoreMesh(
        core_axis_name="c", subcore_axis_name="s",
        num_cores=NUM_CORES, num_subcores=NUM_SUBCORES)

    run = pl.kernel(
        functools.partial(_emb_kernel, n_chunks),
        out_type=jax.ShapeDtypeStruct((b, PAD_D), jnp.float32),
        mesh=mesh,
        scratch_types=[
            pltpu.VMEM((n_chunks, CHUNK), jnp.int32),
            [pltpu.VMEM((CHUNK, PAD_D), jnp.float32) for _ in range(NBUF)],
            [pltpu.SemaphoreType.DMA for _ in range(NBUF)],
            [pltpu.SemaphoreType.DMA for _ in range(NBUF)],
        ],
        compiler_params=pltpu.CompilerParams(use_tc_tiling_on_sc=False),
    )
    out = run(flat_ids, wpad)
    return out[:, :d].reshape(batch, seq, d)
